# baseline (speedup 1.0000x reference)
import functools

import jax
import jax.numpy as jnp
from jax import lax
from jax.experimental import pallas as pl
from jax.experimental.pallas import tpu as pltpu
from jax.experimental.pallas import tpu_sc as plsc

BATCH = 4096
SEQ = 200
VOCAB = 1000000
EMB = 64
HIDDEN = 256
CLASSES = 16

NUM_CORES = 2
NUM_SUBCORES = 16
LANES = 16
NW = NUM_CORES * NUM_SUBCORES
ROWS_PER_W = BATCH // NW
NBUF = 4
SPLIT = 128
REST = SEQ - SPLIT

FULL_COLS = VOCAB // 128
TAIL_ROWS = VOCAB - FULL_COLS * 128
COLS_BASE = FULL_COLS // NW
COLS_EXTRA = FULL_COLS - COLS_BASE * NW

_mesh = plsc.VectorSubcoreMesh(
    core_axis_name="c", subcore_axis_name="s",
    num_cores=NUM_CORES, num_subcores=NUM_SUBCORES)


@functools.partial(
    pl.kernel,
    mesh=_mesh,
    compiler_params=pltpu.CompilerParams(use_tc_tiling_on_sc=True,
                                         needs_layout_passes=False),
    out_type=jax.ShapeDtypeStruct((VOCAB * EMB,), jnp.float32),
    scratch_types=[
        pltpu.VMEM((2, EMB, 128), jnp.float32),
        pltpu.VMEM((2 * 128 * EMB,), jnp.float32),
        pltpu.VMEM((TAIL_ROWS * EMB,), jnp.float32),
        [pltpu.SemaphoreType.DMA] * 2,
        [pltpu.SemaphoreType.DMA] * 2,
    ],
)
def _repack(tblt_hbm, tail_hbm, out_hbm, in_v, stage_v, tail_v,
            in_sems, out_sems):
    wid = lax.axis_index("s") * NUM_CORES + lax.axis_index("c")
    start = wid * COLS_BASE + lax.min(wid, COLS_EXTRA)
    count = COLS_BASE + jnp.where(wid < COLS_EXTRA, 1, 0)

    def issue_in(c, slot):
        pltpu.make_async_copy(
            tblt_hbm.at[:, pl.ds(c * 128, 128)],
            in_v.at[slot],
            in_sems[slot]).start()

    def wait_in(slot):
        pltpu.make_async_copy(
            tblt_hbm.at[:, pl.ds(0, 128)], in_v.at[slot],
            in_sems[slot]).wait()

    def wait_out(slot):
        pltpu.make_async_copy(
            stage_v.at[pl.ds(slot * 128 * EMB, 128 * EMB)],
            out_hbm.at[pl.ds(0, 128 * EMB)],
            out_sems[slot]).wait()

    iota16 = lax.iota(jnp.int32, LANES)

    def transpose_col(i, c, slot):
        def row_body(l, _):
            lsplat = jnp.full((LANES,), l, jnp.int32)
            for a in range(EMB // LANES):
                v = plsc.load_gather(
                    in_v.at[slot], [iota16 + LANES * a, lsplat])
                stage_v[pl.ds(slot * 128 * EMB + l * EMB + LANES * a,
                              LANES)] = v
            return 0
        lax.fori_loop(0, 128, row_body, 0)
        pltpu.make_async_copy(
            stage_v.at[pl.ds(slot * 128 * EMB, 128 * EMB)],
            out_hbm.at[pl.ds(c * 128 * EMB, 128 * EMB)],
            out_sems[slot]).start()

    @pl.when(count > 0)
    def _():
        issue_in(start, 0)

    @pl.when(count > 1)
    def _():
        issue_in(start + 1, 1)

    def outer(j, _):
        for b in range(2):
            i = j * 2 + b

            @pl.when(i < count)
            def _():
                wait_in(b)

                @pl.when(i >= 2)
                def _():
                    wait_out(b)

                transpose_col(i, start + i, b)

                @pl.when(i + 2 < count)
                def _():
                    issue_in(start + i + 2, b)
        return 0

    lax.fori_loop(0, (COLS_BASE + 2) // 2, outer, 0)
    wait_out(0)
    wait_out(1)

    @pl.when(wid == 0)
    def _():
        pltpu.sync_copy(tail_hbm, tail_v)
        pltpu.sync_copy(
            tail_v, out_hbm.at[pl.ds(FULL_COLS * 128 * EMB,
                                     TAIL_ROWS * EMB)])


@functools.partial(
    pl.kernel,
    mesh=_mesh,
    compiler_params=pltpu.CompilerParams(use_tc_tiling_on_sc=False),
    out_type=jax.ShapeDtypeStruct((BATCH, EMB), jnp.float32),
    scratch_types=[
        pltpu.VMEM((ROWS_PER_W, SEQ), jnp.int32),
        pltpu.VMEM((NBUF, SEQ, EMB), jnp.float32),
        pltpu.VMEM((ROWS_PER_W, EMB), jnp.float32),
        [pltpu.SemaphoreType.DMA] * NBUF,
    ],
)
def _pool(x_hbm, table_hbm, out_hbm, idx_v, rows_v, pool_v, sems):
    wid = lax.axis_index("s") * NUM_CORES + lax.axis_index("c")
    base = wid * ROWS_PER_W

    pltpu.sync_copy(x_hbm.at[pl.ds(base, ROWS_PER_W)], idx_v)

    def issue(r, slot):
        pltpu.make_async_copy(
            table_hbm.at[idx_v.at[r, pl.ds(0, SPLIT)]],
            rows_v.at[slot, pl.ds(0, SPLIT)],
            sems[slot]).start()
        pltpu.make_async_copy(
            table_hbm.at[idx_v.at[r, pl.ds(SPLIT, REST)]],
            rows_v.at[slot, pl.ds(SPLIT, REST)],
            sems[slot]).start()

    def wait_slot(slot):
        pltpu.make_async_copy(
            table_hbm.at[pl.ds(0, SEQ)], rows_v.at[slot], sems[slot]).wait()

    def reduce_row(slot, r):
        def body(i, accs):
            return tuple(accs[c] + rows_v[slot, i, pl.ds(LANES * c, LANES)]
                         for c in range(EMB // LANES))
        zero = jnp.zeros((LANES,), jnp.float32)
        accs = lax.fori_loop(0, SEQ, body, (zero,) * (EMB // LANES),
                             unroll=8)
        for c in range(EMB // LANES):
            pool_v[r, pl.ds(LANES * c, LANES)] = accs[c] * (1.0 / SEQ)

    for p in range(NBUF - 1):
        issue(p, p)

    def outer(g, _):
        for b in range(NBUF):
            r = g * NBUF + b
            nxt = r + NBUF - 1

            @pl.when(nxt < ROWS_PER_W)
            def _():
                issue(nxt, (b + NBUF - 1) % NBUF)

            wait_slot(b)
            reduce_row(b, r)
        return 0

    lax.fori_loop(0, ROWS_PER_W // NBUF, outer, 0)
    pltpu.sync_copy(pool_v, out_hbm.at[pl.ds(base, ROWS_PER_W)])


def _mlp_body(x_ref, w1_ref, b1_ref, w2_ref, b2_ref, o_ref):
    h = jnp.dot(x_ref[...], w1_ref[...],
                preferred_element_type=jnp.float32) + b1_ref[...]
    logits = jnp.dot(h, w2_ref[...],
                     preferred_element_type=jnp.float32) + b2_ref[...]
    m = jnp.max(logits, axis=-1, keepdims=True)
    e = jnp.exp(logits - m)
    o_ref[...] = e / jnp.sum(e, axis=-1, keepdims=True)


_mlp = pl.pallas_call(
    _mlp_body,
    out_shape=jax.ShapeDtypeStruct((BATCH, CLASSES), jnp.float32),
)


def kernel(x, emb_table, W1, b1, W2, b2):
    xi = x.astype(jnp.int32)
    tblt = emb_table.T
    tail = emb_table[FULL_COLS * 128:].reshape(-1)
    packed = _repack(tblt, tail)
    pooled = _pool(xi, packed.reshape(VOCAB, EMB))
    return _mlp(pooled, W1, b1.reshape(1, HIDDEN), W2, b2.reshape(1, CLASSES))
